# baseline (speedup 1.0000x reference)
import functools

import jax
import jax.numpy as jnp
from jax import lax
from jax.experimental import pallas as pl
from jax.experimental.pallas import tpu as pltpu
from jax.experimental.pallas import tpu_sc as plsc

NUM_CLASSES = 100000
FEAT_DIM = 64
BATCH = 16384

NC, NS, L = 2, 16, 16
NROUND = 2
SEG = 9984
NSEGP = 10
TAIL = NUM_CLASSES - NSEGP * SEG
CHB = 1024
NCHB = BATCH // CHB

_mesh = plsc.VectorSubcoreMesh(core_axis_name="c", subcore_axis_name="s")


@functools.partial(
    pl.kernel,
    out_type=jax.ShapeDtypeStruct((FEAT_DIM * BATCH,), jnp.float32),
    mesh=_mesh,
    scratch_types=[
        pltpu.VMEM((NUM_CLASSES,), jnp.float32),
        pltpu.VMEM((BATCH,), jnp.int32),
        pltpu.VMEM((2, CHB), jnp.float32),
        pltpu.VMEM_SHARED((2, 8, SEG), jnp.float32),
        pltpu.SemaphoreType.DMA,
        pltpu.SemaphoreType.DMA,
        pltpu.SemaphoreType.DMA,
    ],
    compiler_params=pltpu.CompilerParams(
        needs_layout_passes=False, use_tc_tiling_on_sc=True),
)
def _gather_kernel(labels_hbm, ct_hbm, tail_hbm, out_hbm,
                   crow_v, lab_v, g_v, cstage,
                   csem, wsem0, wsem1):
    cid = lax.axis_index("c")
    sid = lax.axis_index("s")
    is_stager = sid == 0
    f0 = cid * (NROUND * NS)
    wsems = (wsem0, wsem1)

    pltpu.sync_copy(labels_hbm, lab_v)

    for r in range(NROUND):
        fglob = f0 + r * NS + sid
        obase = fglob * BATCH
        pltpu.sync_copy(tail_hbm.at[pl.ds(fglob * TAIL, TAIL)],
                        crow_v.at[pl.ds(NSEGP * SEG, TAIL)])

        def fire_step(s):
            blk8, p = s // NSEGP, s % NSEGP
            return pltpu.async_copy(
                ct_hbm.at[pl.ds(f0 + r * NS + blk8 * 8, 8),
                          pl.ds(p * SEG, SEG)],
                cstage.at[s % 2], csem)

        @pl.when(is_stager)
        def _():
            fire_step(0).wait()

        plsc.subcore_barrier()

        for s in range(2 * NSEGP):
            blk8, p = s // NSEGP, s % NSEGP

            @pl.when(is_stager)
            def _():
                if s + 1 < 2 * NSEGP:
                    fire_step(s + 1).wait()

            @pl.when(sid // 8 == blk8)
            def _():
                pltpu.sync_copy(cstage.at[s % 2, sid % 8],
                                crow_v.at[pl.ds(p * SEG, SEG)])

            plsc.subcore_barrier()

        for k in range(NCHB):
            pb = k % 2
            if k >= 2 or (r > 0 and k < 2):
                pltpu.make_async_copy(g_v.at[pb],
                                      out_hbm.at[pl.ds(0, CHB)],
                                      wsems[pb]).wait()

            kbase = k * CHB

            def blk_body(blk, _):
                for u in range(4):
                    off = blk * (4 * L) + u * L
                    lab = lab_v[pl.ds(kbase + off, L)]
                    g_v[pb, pl.ds(off, L)] = plsc.load_gather(crow_v, [lab])
                return 0

            lax.fori_loop(0, CHB // (4 * L), blk_body, 0)
            pltpu.async_copy(g_v.at[pb],
                             out_hbm.at[pl.ds(obase + k * CHB, CHB)],
                             wsems[pb])

    for pb in range(2):
        pltpu.make_async_copy(g_v.at[pb], out_hbm.at[pl.ds(0, CHB)],
                              wsems[pb]).wait()


def _reduce_body(g_ref, x_ref, o_ref):
    d = g_ref[...] - x_ref[...]
    s = jnp.sum(d * d, axis=0)
    o_ref[...] = jnp.sum(jnp.clip(s, 1e-12, 1e12)).reshape(1, 1)


def _reduce(g2, xt):
    return pl.pallas_call(
        _reduce_body,
        out_shape=jax.ShapeDtypeStruct((1, 1), jnp.float32),
    )(g2, xt)


def kernel(x, labels, centers):
    ct = centers.T
    tail = ct[:, NSEGP * SEG:].reshape(-1)
    g = _gather_kernel(labels.astype(jnp.int32), ct, tail)
    g2 = g.reshape(FEAT_DIM, BATCH)
    return _reduce(g2, x.T)[0, 0] / BATCH


# speedup vs baseline: 1.1563x; 1.1563x over previous
import functools

import jax
import jax.numpy as jnp
from jax import lax
from jax.experimental import pallas as pl
from jax.experimental.pallas import tpu as pltpu
from jax.experimental.pallas import tpu_sc as plsc

NUM_CLASSES = 100000
FEAT_DIM = 64
BATCH = 16384

NC, NS, L = 2, 16, 16
NROUND = 2
SEG = 19968
NSEGP = 5
TAIL = NUM_CLASSES - NSEGP * SEG
CHB = 1024
NCHB = BATCH // CHB

_mesh = plsc.VectorSubcoreMesh(core_axis_name="c", subcore_axis_name="s")


@functools.partial(
    pl.kernel,
    out_type=jax.ShapeDtypeStruct((FEAT_DIM * BATCH,), jnp.float32),
    mesh=_mesh,
    scratch_types=[
        pltpu.VMEM((NUM_CLASSES,), jnp.float32),
        pltpu.VMEM((4, CHB), jnp.int32),
        pltpu.VMEM((2, CHB), jnp.float32),
        pltpu.VMEM_SHARED((2, 8, SEG), jnp.float32),
        pltpu.SemaphoreType.DMA,
        pltpu.SemaphoreType.DMA,
        pltpu.SemaphoreType.DMA,
        pltpu.SemaphoreType.DMA,
        pltpu.SemaphoreType.DMA,
        pltpu.SemaphoreType.DMA,
        pltpu.SemaphoreType.DMA,
    ],
    compiler_params=pltpu.CompilerParams(
        needs_layout_passes=False, use_tc_tiling_on_sc=True),
)
def _gather_kernel(labels_hbm, ct_hbm, tail_hbm, out_hbm,
                   crow_v, lab_v, g_v, cstage,
                   csem, lsem0, lsem1, lsem2, lsem3, wsem0, wsem1):
    lsems = (lsem0, lsem1, lsem2, lsem3)
    cid = lax.axis_index("c")
    sid = lax.axis_index("s")
    is_stager = sid == 0
    f0 = cid * (NROUND * NS)
    wsems = (wsem0, wsem1)


    for r in range(NROUND):
        fglob = f0 + r * NS + sid
        obase = fglob * BATCH
        pltpu.sync_copy(tail_hbm.at[pl.ds(fglob * TAIL, TAIL)],
                        crow_v.at[pl.ds(NSEGP * SEG, TAIL)])

        def fire_step(s):
            blk8, p = s // NSEGP, s % NSEGP
            return pltpu.async_copy(
                ct_hbm.at[pl.ds(f0 + r * NS + blk8 * 8, 8),
                          pl.ds(p * SEG, SEG)],
                cstage.at[s % 2], csem)

        @pl.when(is_stager)
        def _():
            fire_step(0).wait()

        plsc.subcore_barrier()

        for s in range(2 * NSEGP):
            blk8, p = s // NSEGP, s % NSEGP

            @pl.when(is_stager)
            def _():
                if s + 1 < 2 * NSEGP:
                    fire_step(s + 1).wait()

            @pl.when(sid // 8 == blk8)
            def _():
                pltpu.sync_copy(cstage.at[s % 2, sid % 8],
                                crow_v.at[pl.ds(p * SEG, SEG)])

            plsc.subcore_barrier()

        def fire_lab(k):
            lb = k % 4
            pltpu.async_copy(labels_hbm.at[pl.ds(k * CHB, CHB)],
                             lab_v.at[lb], lsems[lb])

        if r == 0:
            for k in range(3):
                fire_lab(k)
        for k in range(NCHB):
            pb = k % 2
            lb = k % 4
            if k + 3 < NCHB:
                fire_lab(k + 3)
            elif r == 0 and k >= NCHB - 3:
                fire_lab(k + 3 - NCHB)
            pltpu.make_async_copy(labels_hbm.at[pl.ds(0, CHB)],
                                  lab_v.at[lb], lsems[lb]).wait()
            if k >= 2 or (r > 0 and k < 2):
                pltpu.make_async_copy(g_v.at[pb],
                                      out_hbm.at[pl.ds(0, CHB)],
                                      wsems[pb]).wait()

            def blk_body(blk, _):
                for u in range(4):
                    off = blk * (4 * L) + u * L
                    lab = lab_v[lb, pl.ds(off, L)]
                    g_v[pb, pl.ds(off, L)] = plsc.load_gather(crow_v, [lab])
                return 0

            lax.fori_loop(0, CHB // (4 * L), blk_body, 0)
            pltpu.async_copy(g_v.at[pb],
                             out_hbm.at[pl.ds(obase + k * CHB, CHB)],
                             wsems[pb])

    for pb in range(2):
        pltpu.make_async_copy(g_v.at[pb], out_hbm.at[pl.ds(0, CHB)],
                              wsems[pb]).wait()


def _reduce_body(g_ref, x_ref, o_ref):
    d = g_ref[...] - x_ref[...]
    s = jnp.sum(d * d, axis=0)
    o_ref[...] = jnp.sum(jnp.clip(s, 1e-12, 1e12)).reshape(1, 1)


def _reduce(g2, xt):
    return pl.pallas_call(
        _reduce_body,
        out_shape=jax.ShapeDtypeStruct((1, 1), jnp.float32),
    )(g2, xt)


def kernel(x, labels, centers):
    ct = centers.T
    tail = ct[:, NSEGP * SEG:].reshape(-1)
    g = _gather_kernel(labels.astype(jnp.int32), ct, tail)
    g2 = g.reshape(FEAT_DIM, BATCH)
    return _reduce(g2, x.T)[0, 0] / BATCH


# speedup vs baseline: 1.3892x; 1.2014x over previous
import functools

import jax
import jax.numpy as jnp
from jax import lax
from jax.experimental import pallas as pl
from jax.experimental.pallas import tpu as pltpu
from jax.experimental.pallas import tpu_sc as plsc

NUM_CLASSES = 100000
FEAT_DIM = 64
BATCH = 16384

NC, NS, L = 2, 16, 16
NW = NC * NS
BPW = BATCH // NW
GROUPS = BPW // L
CH = 256
NCH = BPW // CH

_mesh = plsc.VectorSubcoreMesh(core_axis_name="c", subcore_axis_name="s")


@functools.partial(
    pl.kernel,
    out_type=jax.ShapeDtypeStruct((NW, L), jnp.float32),
    mesh=_mesh,
    scratch_types=[
        pltpu.VMEM((BPW,), jnp.int32),
        pltpu.SMEM((BPW,), jnp.int32),
        pltpu.VMEM((CH, FEAT_DIM), jnp.float32),
        pltpu.VMEM((CH, FEAT_DIM), jnp.float32),
        pltpu.VMEM((L,), jnp.float32),
        pltpu.SemaphoreType.DMA,
        pltpu.SemaphoreType.DMA,
        pltpu.SemaphoreType.DMA,
    ],
    compiler_params=pltpu.CompilerParams(
        needs_layout_passes=False, use_tc_tiling_on_sc=True),
)
def _center_loss_kernel(x_hbm, labels_hbm, centers_hbm, out_hbm,
                        idx_v, idx_s, c_v, x_v, part_v, gsem, xsem, isem):
    wid = lax.axis_index("s") * NC + lax.axis_index("c")
    base = wid * BPW

    pltpu.sync_copy(labels_hbm.at[pl.ds(base, BPW)], idx_v)

    lane = lax.iota(jnp.int32, L)

    def group_body(g, tot):
        rows = g * L + lane
        accs = [jnp.zeros((L,), jnp.float32) for _ in range(4)]
        for f in range(FEAT_DIM):
            col = (lane + f) & (FEAT_DIM - 1)
            c = plsc.load_gather(c_v, [rows, col])
            xv = plsc.load_gather(x_v, [rows, col])
            d = c - xv
            accs[f % 4] = accs[f % 4] + d * d
        acc = (accs[0] + accs[1]) + (accs[2] + accs[3])
        acc = jnp.clip(acc, 1e-12, 1e12)
        return tot + acc

    tot = jnp.zeros((L,), jnp.float32)
    for ch in range(NCH):
        xcopy = pltpu.async_copy(
            x_hbm.at[pl.ds(base + ch * CH, CH)], x_v, xsem)

        def fire(blk, _):
            vec = idx_v[pl.ds(ch * CH + blk * L, L)]
            for j in range(L):
                pltpu.async_copy(
                    centers_hbm.at[vec[j]], c_v.at[blk * L + j], gsem)
            return 0

        lax.fori_loop(0, CH // L, fire, 0)
        pltpu.make_async_copy(x_hbm.at[pl.ds(0, CH)], c_v, gsem).wait()
        xcopy.wait()
        tot = lax.fori_loop(0, CH // L, group_body, tot)
    part_v[...] = tot
    pltpu.sync_copy(part_v, out_hbm.at[wid])


def kernel(x, labels, centers):
    labels1 = labels.astype(jnp.int32)
    parts = _center_loss_kernel(x, labels1, centers)
    return jnp.sum(parts) / BATCH


# speedup vs baseline: 1.4124x; 1.0167x over previous
import functools

import jax
import jax.numpy as jnp
from jax import lax
from jax.experimental import pallas as pl
from jax.experimental.pallas import tpu as pltpu
from jax.experimental.pallas import tpu_sc as plsc

NUM_CLASSES = 100000
FEAT_DIM = 64
BATCH = 16384

NC, NS, L = 2, 16, 16
NW = NC * NS
BPW = BATCH // NW
GROUPS = BPW // L
CH = 128
NCH = BPW // CH

_mesh = plsc.VectorSubcoreMesh(core_axis_name="c", subcore_axis_name="s")


@functools.partial(
    pl.kernel,
    out_type=jax.ShapeDtypeStruct((NW, L), jnp.float32),
    mesh=_mesh,
    scratch_types=[
        pltpu.VMEM((BPW,), jnp.int32),
        pltpu.VMEM((2, CH, FEAT_DIM), jnp.float32),
        pltpu.VMEM((2, CH, FEAT_DIM), jnp.float32),
        pltpu.VMEM((L,), jnp.float32),
        pltpu.SemaphoreType.DMA,
        pltpu.SemaphoreType.DMA,
        pltpu.SemaphoreType.DMA,
        pltpu.SemaphoreType.DMA,
    ],
    compiler_params=pltpu.CompilerParams(
        needs_layout_passes=False, use_tc_tiling_on_sc=True),
)
def _center_loss_kernel(x_hbm, labels_hbm, centers_hbm, out_hbm,
                        idx_v, c_v, x_v, part_v, gsem0, gsem1, xsem0, xsem1):
    gsems = (gsem0, gsem1)
    xsems = (xsem0, xsem1)
    wid = lax.axis_index("s") * NC + lax.axis_index("c")
    base = wid * BPW

    pltpu.sync_copy(labels_hbm.at[pl.ds(base, BPW)], idx_v)

    lane = lax.iota(jnp.int32, L)

    def make_group_body(pb):
        def group_body(g, tot):
            rows = g * L + lane
            accs = [jnp.zeros((L,), jnp.float32) for _ in range(4)]
            for f in range(FEAT_DIM):
                col = (lane + f) & (FEAT_DIM - 1)
                c = plsc.load_gather(c_v.at[pb], [rows, col])
                xv = plsc.load_gather(x_v.at[pb], [rows, col])
                d = c - xv
                accs[f % 4] = accs[f % 4] + d * d
            acc = (accs[0] + accs[1]) + (accs[2] + accs[3])
            acc = jnp.clip(acc, 1e-12, 1e12)
            return tot + acc
        return group_body

    def unused_group_body(g, tot):
        rows = g * L + lane
        accs = [jnp.zeros((L,), jnp.float32) for _ in range(4)]
        for f in range(FEAT_DIM):
            col = (lane + f) & (FEAT_DIM - 1)
            c = plsc.load_gather(c_v, [rows, col])
            xv = plsc.load_gather(x_v, [rows, col])
            d = c - xv
            accs[f % 4] = accs[f % 4] + d * d
        acc = (accs[0] + accs[1]) + (accs[2] + accs[3])
        acc = jnp.clip(acc, 1e-12, 1e12)
        return tot + acc

    def fire_chunk(ch):
        pb = ch % 2
        pltpu.async_copy(
            x_hbm.at[pl.ds(base + ch * CH, CH)], x_v.at[pb], xsems[pb])

        def fire(blk, _):
            vec = idx_v[pl.ds(ch * CH + blk * L, L)]
            for j in range(L):
                pltpu.async_copy(
                    centers_hbm.at[vec[j]], c_v.at[pb, blk * L + j],
                    gsems[pb])
            return 0

        lax.fori_loop(0, CH // L, fire, 0)

    tot = jnp.zeros((L,), jnp.float32)
    fire_chunk(0)
    for ch in range(NCH):
        pb = ch % 2
        if ch + 1 < NCH:
            fire_chunk(ch + 1)
        pltpu.make_async_copy(x_hbm.at[pl.ds(0, CH)], c_v.at[pb],
                              gsems[pb]).wait()
        pltpu.make_async_copy(x_hbm.at[pl.ds(0, CH)], x_v.at[pb],
                              xsems[pb]).wait()
        tot = lax.fori_loop(0, CH // L, make_group_body(pb), tot)
    part_v[...] = tot
    pltpu.sync_copy(part_v, out_hbm.at[wid])


def kernel(x, labels, centers):
    labels1 = labels.astype(jnp.int32)
    parts = _center_loss_kernel(x, labels1, centers)
    return jnp.sum(parts) / BATCH
